# Initial kernel scaffold; baseline (speedup 1.0000x reference)
import jax
import jax.numpy as jnp
from jax.experimental import pallas as pl


def kernel(x, x_mask, edge_attr, pos, sampling_points, W_enc, b_enc, W_pos, b_pos, W_msg, b_msg, W_upd, b_upd, W_dec, b_dec, edge_index, batch):
    raise NotImplementedError("write your pallas kernel here")


# speedup vs baseline: 3.6851x; 3.6851x over previous
import functools

import jax
import jax.numpy as jnp
from jax import lax
from jax.experimental import pallas as pl
from jax.experimental.pallas import tpu as pltpu
from jax.experimental.pallas import tpu_sc as plsc

G = 16
NC = 2
NS = 16
CHUNK = 64


def _pre_body(h0, batch2, batch_r, bc2, pos, W_enc, b_enc, Wm_s, Wm_d, Wm_p,
              W3, W4, b_upd,
              h_o, P_o, oh_o, ohnt_o, crow_o, xbc4b_o, A_o, B_o):
    f32 = jnp.float32
    h = jnp.maximum(jnp.dot(h0[...], W_enc[...],
                            preferred_element_type=f32) + b_enc[...], 0.0)
    n = h.shape[0]
    iota_cols = lax.broadcasted_iota(jnp.int32, (n, G), 1)
    oh = (batch2[...] == iota_cols).astype(f32)
    iota_rows = lax.broadcasted_iota(jnp.int32, (G, n), 0)
    oht = (batch_r[...] == iota_rows).astype(f32)
    cnt = jnp.sum(oht, axis=1, keepdims=True)
    ohnt = oht / jnp.maximum(cnt, 1.0)
    bc = (bc2[...] > 0.5).astype(f32)
    ohbct = oht * jnp.reshape(bc, (1, n))
    cnt_bc = jnp.maximum(jnp.sum(ohbct, axis=1, keepdims=True), 1.0)
    x_bc = jnp.dot(ohbct, h, preferred_element_type=f32) / cnt_bc
    xg = jnp.dot(ohnt, h, preferred_element_type=f32)
    xbc4b = jnp.dot(x_bc, W4[...], preferred_element_type=f32) + b_upd[...]
    crow = jnp.dot(xg, W3[...], preferred_element_type=f32) + xbc4b
    P = jnp.dot(pos[...], Wm_p[...], preferred_element_type=f32)
    h_o[...] = h
    P_o[...] = P
    oh_o[...] = oh
    ohnt_o[...] = ohnt
    crow_o[...] = crow
    xbc4b_o[...] = xbc4b
    A_o[...] = jnp.dot(h, Wm_s[...], preferred_element_type=f32) - P
    B_o[...] = jnp.dot(h, Wm_d[...], preferred_element_type=f32) + P


def _c_body(ea, Wm_e, b_msg, c_o):
    c_o[...] = (jnp.dot(ea[...], Wm_e[...], preferred_element_type=jnp.float32)
                + b_msg[...])


def _upd_body(h, a0, a1, d0, d1, oh, ohnt, crow, P, xbc4b,
              W1, W2, Wm_s, Wm_d, W3,
              h_o, A_o, B_o, crow_o):
    f32 = jnp.float32
    deg = jnp.maximum(d0[...] + d1[...], 1.0)
    agg = (a0[...] + a1[...]) / deg
    u = jnp.maximum(
        jnp.dot(h[...], W1[...], preferred_element_type=f32)
        + jnp.dot(agg, W2[...], preferred_element_type=f32)
        + jnp.dot(oh[...], crow[...], preferred_element_type=f32), 0.0)
    h2 = h[...] + u
    xg = jnp.dot(ohnt[...], h2, preferred_element_type=f32)
    h_o[...] = h2
    A_o[...] = jnp.dot(h2, Wm_s[...], preferred_element_type=f32) - P[...]
    B_o[...] = jnp.dot(h2, Wm_d[...], preferred_element_type=f32) + P[...]
    crow_o[...] = jnp.dot(xg, W3[...], preferred_element_type=f32) + xbc4b[...]


def _epi_body(h, sp, W_dec, b_dec, W_pos, b_pos, u_o, nodes_o):
    f32 = jnp.float32
    nodes_o[...] = (jnp.dot(h[...], W_dec[...], preferred_element_type=f32)
                    + b_dec[...])
    es = jnp.maximum(jnp.dot(sp[...], W_pos[...],
                             preferred_element_type=f32) + b_pos[...], 0.0)
    u_o[...] = jnp.dot(es, W_dec[...], preferred_element_type=f32) + b_dec[...]


def _chunk_ranges(n_edges):
    total_chunks = n_edges // CHUNK
    cpw = -(-total_chunks // (NC * NS))
    return total_chunks, cpw


def _edge_sc_body(n_pad, n_edges,
                  A2, B2, C, src, dst, z128,
                  agg_o,
                  si_v, di_v, a_v, b_v, c_v,
                  acc_sh, sem_a, sem_b, sem_c):
    cid = lax.axis_index("c")
    sid = lax.axis_index("s")
    wid = sid * NC + cid
    rpt = n_pad // NS
    r0 = sid * rpt

    total_chunks, cpw = _chunk_ranges(n_edges)
    start = wid * cpw
    n_my = jnp.maximum(jnp.minimum(cpw, total_chunks - start), 0)

    pltpu.sync_copy(z128.at[pl.ds(r0, rpt)], acc_sh.at[pl.ds(r0, rpt)])
    plsc.subcore_barrier()

    def chunk_body(i, carry):
        ebase = (start + i) * CHUNK
        pltpu.sync_copy(src.at[pl.ds(ebase, CHUNK)], si_v)
        pltpu.sync_copy(dst.at[pl.ds(ebase, CHUNK)], di_v)
        ca = pltpu.async_copy(A2.at[si_v], a_v, sem_a)
        cb = pltpu.async_copy(B2.at[di_v], b_v, sem_b)
        cc = pltpu.async_copy(C.at[pl.ds(ebase, CHUNK)], c_v, sem_c)
        ca.wait()
        cb.wait()
        cc.wait()

        def row_body(r, carry2):
            for j in range(8):
                s = pl.ds(j * 16, 16)
                c_v[r, s] = jnp.maximum(a_v[r, s] + b_v[r, s] + c_v[r, s], 0.0)
            return carry2
        lax.fori_loop(0, CHUNK, row_body, 0, unroll=False)

        pltpu.sync_copy(c_v, acc_sh.at[di_v], add=True)
        return carry
    lax.fori_loop(0, n_my, chunk_body, 0, unroll=False)

    plsc.subcore_barrier()
    pltpu.sync_copy(acc_sh.at[pl.ds(r0, rpt)],
                    agg_o.at[pl.ds(cid * n_pad + r0, rpt)])


def _make_edge_kernel(n_pad, n_edges):
    mesh = plsc.VectorSubcoreMesh(core_axis_name="c", subcore_axis_name="s")
    return pl.kernel(
        functools.partial(_edge_sc_body, n_pad, n_edges),
        out_type=jax.ShapeDtypeStruct((NC * n_pad, 128), jnp.float32),
        mesh=mesh,
        scratch_types=[
            pltpu.VMEM((CHUNK,), jnp.int32),
            pltpu.VMEM((CHUNK,), jnp.int32),
            pltpu.VMEM((CHUNK, 128), jnp.float32),
            pltpu.VMEM((CHUNK, 128), jnp.float32),
            pltpu.VMEM((CHUNK, 128), jnp.float32),
            pltpu.VMEM_SHARED((n_pad, 128), jnp.float32),
            pltpu.SemaphoreType.DMA,
            pltpu.SemaphoreType.DMA,
            pltpu.SemaphoreType.DMA,
        ],
        name="edge_messages_sc",
    )


def _deg_sc_body(n_pad, n_edges,
                 dst, z128,
                 deg_o,
                 di_v, ones_v, acc_sh):
    cid = lax.axis_index("c")
    sid = lax.axis_index("s")
    wid = sid * NC + cid
    rpt = n_pad // NS
    r0 = sid * rpt

    total_chunks, cpw = _chunk_ranges(n_edges)
    start = wid * cpw
    n_my = jnp.maximum(jnp.minimum(cpw, total_chunks - start), 0)

    pltpu.sync_copy(z128.at[pl.ds(r0, rpt)], acc_sh.at[pl.ds(r0, rpt)])
    one = jnp.ones((16,), jnp.float32)

    def fill_body(r, carry):
        for j in range(8):
            ones_v[r, pl.ds(j * 16, 16)] = one
        return carry
    lax.fori_loop(0, CHUNK, fill_body, 0, unroll=False)
    plsc.subcore_barrier()

    def chunk_body(i, carry):
        ebase = (start + i) * CHUNK
        pltpu.sync_copy(dst.at[pl.ds(ebase, CHUNK)], di_v)
        pltpu.sync_copy(ones_v, acc_sh.at[di_v], add=True)
        return carry
    lax.fori_loop(0, n_my, chunk_body, 0, unroll=False)

    plsc.subcore_barrier()
    pltpu.sync_copy(acc_sh.at[pl.ds(r0, rpt)],
                    deg_o.at[pl.ds(cid * n_pad + r0, rpt)])


def _make_deg_kernel(n_pad, n_edges):
    mesh = plsc.VectorSubcoreMesh(core_axis_name="c", subcore_axis_name="s")
    return pl.kernel(
        functools.partial(_deg_sc_body, n_pad, n_edges),
        out_type=jax.ShapeDtypeStruct((NC * n_pad, 128), jnp.float32),
        mesh=mesh,
        scratch_types=[
            pltpu.VMEM((CHUNK,), jnp.int32),
            pltpu.VMEM((CHUNK, 128), jnp.float32),
            pltpu.VMEM_SHARED((n_pad, 128), jnp.float32),
        ],
        name="degree_sc",
    )


REPEATS = 4


def kernel(x, x_mask, edge_attr, pos, sampling_points,
           W_enc, b_enc, W_pos, b_pos, W_msg, b_msg, W_upd, b_upd,
           W_dec, b_dec, edge_index, batch):
    f32 = jnp.float32
    n = x.shape[0]
    e = edge_index.shape[1]
    H = W_enc.shape[1]

    h0 = jnp.concatenate([x, x_mask], axis=1)
    batch2 = batch[:, None]
    batch_r = batch[None, :]
    bc2 = x_mask[:, 1:2]
    src = edge_index[0]
    dst = edge_index[1]
    Wm_s = W_msg[:H]
    Wm_d = W_msg[H:2 * H]
    Wm_e = W_msg[2 * H:2 * H + 4]
    Wm_p = W_msg[2 * H + 4:]
    W1 = W_upd[:H]
    W2 = W_upd[H:2 * H]
    W3 = W_upd[2 * H:3 * H]
    W4 = W_upd[3 * H:]
    n_pad = -(-n // (8 * NS)) * (8 * NS)
    z128 = jnp.zeros((n_pad, 128), f32)

    nf = jax.ShapeDtypeStruct((n, H), f32)
    gf = jax.ShapeDtypeStruct((G, H), f32)
    h, P, oh, ohnt, crow, xbc4b, A2, B2 = pl.pallas_call(
        _pre_body,
        out_shape=(nf, nf, jax.ShapeDtypeStruct((n, G), f32),
                   jax.ShapeDtypeStruct((G, n), f32), gf, gf, nf, nf),
        name="pre_tc",
    )(h0, batch2, batch_r, bc2, pos, W_enc, b_enc[None, :], Wm_s, Wm_d, Wm_p,
      W3, W4, b_upd[None, :])

    eb = 3200
    C = pl.pallas_call(
        _c_body,
        grid=(e // eb,),
        in_specs=[pl.BlockSpec((eb, 4), lambda i: (i, 0)),
                  pl.BlockSpec((4, H), lambda i: (0, 0)),
                  pl.BlockSpec((1, H), lambda i: (0, 0))],
        out_specs=pl.BlockSpec((eb, H), lambda i: (i, 0)),
        out_shape=jax.ShapeDtypeStruct((e, H), f32),
        name="edge_const_tc",
    )(edge_attr, Wm_e, b_msg[None, :])

    edge_k = _make_edge_kernel(n_pad, e)
    degp = _make_deg_kernel(n_pad, e)(dst, z128)
    d0 = degp[:n, :1]
    d1 = degp[n_pad:n_pad + n, :1]
    upd = pl.pallas_call(
        _upd_body,
        out_shape=(nf, nf, nf, gf),
        name="update_tc",
    )

    for _ in range(REPEATS):
        aggp = edge_k(A2, B2, C, src, dst, z128)
        h, A2, B2, crow = upd(
            h, aggp[:n], aggp[n_pad:n_pad + n], d0, d1,
            oh, ohnt, crow, P, xbc4b, W1, W2, Wm_s, Wm_d, W3)

    U, nodes = pl.pallas_call(
        _epi_body,
        out_shape=(jax.ShapeDtypeStruct((sampling_points.shape[0], 4), f32),
                   jax.ShapeDtypeStruct((n, 4), f32)),
        name="epi_tc",
    )(h, sampling_points, W_dec, b_dec[None, :], W_pos, b_pos[None, :])
    return (U, nodes)


# speedup vs baseline: 5.4145x; 1.4693x over previous
import functools

import jax
import jax.numpy as jnp
from jax import lax
from jax.experimental import pallas as pl
from jax.experimental.pallas import tpu as pltpu
from jax.experimental.pallas import tpu_sc as plsc

G = 16
NC = 2
NS = 16
CHUNK = 64


def _pre_body(h0, batch2, batch_r, bc2, pos, W_enc, b_enc, Wm_s, Wm_d, Wm_p,
              W3, W4, b_upd,
              h_o, P_o, oh_o, ohnt_o, crow_o, xbc4b_o, A_o, B_o):
    f32 = jnp.float32
    h = jnp.maximum(jnp.dot(h0[...], W_enc[...],
                            preferred_element_type=f32) + b_enc[...], 0.0)
    n = h.shape[0]
    iota_cols = lax.broadcasted_iota(jnp.int32, (n, G), 1)
    oh = (batch2[...] == iota_cols).astype(f32)
    iota_rows = lax.broadcasted_iota(jnp.int32, (G, n), 0)
    oht = (batch_r[...] == iota_rows).astype(f32)
    cnt = jnp.sum(oht, axis=1, keepdims=True)
    ohnt = oht / jnp.maximum(cnt, 1.0)
    bc = (bc2[...] > 0.5).astype(f32)
    ohbct = oht * jnp.reshape(bc, (1, n))
    cnt_bc = jnp.maximum(jnp.sum(ohbct, axis=1, keepdims=True), 1.0)
    x_bc = jnp.dot(ohbct, h, preferred_element_type=f32) / cnt_bc
    xg = jnp.dot(ohnt, h, preferred_element_type=f32)
    xbc4b = jnp.dot(x_bc, W4[...], preferred_element_type=f32) + b_upd[...]
    crow = jnp.dot(xg, W3[...], preferred_element_type=f32) + xbc4b
    P = jnp.dot(pos[...], Wm_p[...], preferred_element_type=f32)
    h_o[...] = h
    P_o[...] = P
    oh_o[...] = oh
    ohnt_o[...] = ohnt
    crow_o[...] = crow
    xbc4b_o[...] = xbc4b
    A_o[...] = jnp.dot(h, Wm_s[...], preferred_element_type=f32) - P
    B_o[...] = jnp.dot(h, Wm_d[...], preferred_element_type=f32) + P


def _c_body(ea, Wm_e, b_msg, c_o):
    c_o[...] = (jnp.dot(ea[...], Wm_e[...], preferred_element_type=jnp.float32)
                + b_msg[...])


def _upd_body(h, a0, a1, d0, d1, oh, ohnt, crow, P, xbc4b,
              W1, W2, Wm_s, Wm_d, W3,
              h_o, A_o, B_o, crow_o):
    f32 = jnp.float32
    deg = jnp.maximum(d0[...] + d1[...], 1.0)
    agg = (a0[...] + a1[...]) / deg
    u = jnp.maximum(
        jnp.dot(h[...], W1[...], preferred_element_type=f32)
        + jnp.dot(agg, W2[...], preferred_element_type=f32)
        + jnp.dot(oh[...], crow[...], preferred_element_type=f32), 0.0)
    h2 = h[...] + u
    xg = jnp.dot(ohnt[...], h2, preferred_element_type=f32)
    h_o[...] = h2
    A_o[...] = jnp.dot(h2, Wm_s[...], preferred_element_type=f32) - P[...]
    B_o[...] = jnp.dot(h2, Wm_d[...], preferred_element_type=f32) + P[...]
    crow_o[...] = jnp.dot(xg, W3[...], preferred_element_type=f32) + xbc4b[...]


def _epi_body(h, sp, W_dec, b_dec, W_pos, b_pos, u_o, nodes_o):
    f32 = jnp.float32
    nodes_o[...] = (jnp.dot(h[...], W_dec[...], preferred_element_type=f32)
                    + b_dec[...])
    es = jnp.maximum(jnp.dot(sp[...], W_pos[...],
                             preferred_element_type=f32) + b_pos[...], 0.0)
    u_o[...] = jnp.dot(es, W_dec[...], preferred_element_type=f32) + b_dec[...]


def _chunk_ranges(n_edges):
    total_chunks = n_edges // CHUNK
    cpw = -(-total_chunks // (NC * NS))
    return total_chunks, cpw


def _edge_sc_body(n_pad, n_edges,
                  A2, B2, C, src, dst, z128,
                  agg_o,
                  si_v, di_v, a_v, b_v, c_v,
                  acc_sh, sem_a, sem_b, sem_c, sem_s):
    cid = lax.axis_index("c")
    sid = lax.axis_index("s")
    wid = sid * NC + cid
    rpt = n_pad // NS
    r0 = sid * rpt

    total_chunks, cpw = _chunk_ranges(n_edges)
    start = wid * cpw
    n_my = jnp.maximum(jnp.minimum(cpw, total_chunks - start), 0)

    pltpu.sync_copy(z128.at[pl.ds(r0, rpt)], acc_sh.at[pl.ds(r0, rpt)])
    plsc.subcore_barrier()

    def stage(slot, ci):
        ebase = (start + ci) * CHUNK
        pltpu.sync_copy(src.at[pl.ds(ebase, CHUNK)], si_v.at[slot])
        pltpu.sync_copy(dst.at[pl.ds(ebase, CHUNK)], di_v.at[slot])
        pltpu.async_copy(A2.at[si_v.at[slot]], a_v.at[slot], sem_a.at[slot])
        pltpu.async_copy(B2.at[di_v.at[slot]], b_v.at[slot], sem_b.at[slot])
        pltpu.async_copy(C.at[pl.ds(ebase, CHUNK)], c_v.at[slot], sem_c.at[slot])

    @pl.when(n_my > 0)
    def _():
        stage(0, 0)

    def chunk_body(i, carry):
        p = lax.rem(i, 2)
        q = lax.rem(i + 1, 2)
        ebase = (start + i) * CHUNK

        @pl.when(i >= 1)
        def _():
            pltpu.make_async_copy(c_v.at[q], acc_sh.at[di_v.at[q]],
                                  sem_s.at[q]).wait()

        @pl.when(i + 1 < n_my)
        def _():
            stage(q, i + 1)

        pltpu.make_async_copy(A2.at[si_v.at[p]], a_v.at[p], sem_a.at[p]).wait()
        pltpu.make_async_copy(B2.at[di_v.at[p]], b_v.at[p], sem_b.at[p]).wait()
        pltpu.make_async_copy(C.at[pl.ds(ebase, CHUNK)], c_v.at[p],
                              sem_c.at[p]).wait()

        @plsc.parallel_loop(0, CHUNK, unroll=2)
        def row_body(r):
            for j in range(8):
                s = pl.ds(j * 16, 16)
                c_v[p, r, s] = jnp.maximum(
                    a_v[p, r, s] + b_v[p, r, s] + c_v[p, r, s], 0.0)

        pltpu.async_copy(c_v.at[p], acc_sh.at[di_v.at[p]], sem_s.at[p],
                         add=True)
        return carry
    lax.fori_loop(0, n_my, chunk_body, 0, unroll=False)

    @pl.when(n_my >= 1)
    def _():
        pf = lax.rem(n_my - 1, 2)
        pltpu.make_async_copy(c_v.at[pf], acc_sh.at[di_v.at[pf]],
                              sem_s.at[pf]).wait()

    plsc.subcore_barrier()
    pltpu.sync_copy(acc_sh.at[pl.ds(r0, rpt)],
                    agg_o.at[pl.ds(cid * n_pad + r0, rpt)])


def _make_edge_kernel(n_pad, n_edges):
    mesh = plsc.VectorSubcoreMesh(core_axis_name="c", subcore_axis_name="s")
    return pl.kernel(
        functools.partial(_edge_sc_body, n_pad, n_edges),
        out_type=jax.ShapeDtypeStruct((NC * n_pad, 128), jnp.float32),
        mesh=mesh,
        scratch_types=[
            pltpu.VMEM((2, CHUNK), jnp.int32),
            pltpu.VMEM((2, CHUNK), jnp.int32),
            pltpu.VMEM((2, CHUNK, 128), jnp.float32),
            pltpu.VMEM((2, CHUNK, 128), jnp.float32),
            pltpu.VMEM((2, CHUNK, 128), jnp.float32),
            pltpu.VMEM_SHARED((n_pad, 128), jnp.float32),
            pltpu.SemaphoreType.DMA((2,)),
            pltpu.SemaphoreType.DMA((2,)),
            pltpu.SemaphoreType.DMA((2,)),
            pltpu.SemaphoreType.DMA((2,)),
        ],
        name="edge_messages_sc",
    )


def _deg_sc_body(n_pad, n_edges,
                 dst, z128,
                 deg_o,
                 di_v, ones_v, acc_sh):
    cid = lax.axis_index("c")
    sid = lax.axis_index("s")
    wid = sid * NC + cid
    rpt = n_pad // NS
    r0 = sid * rpt

    total_chunks, cpw = _chunk_ranges(n_edges)
    start = wid * cpw
    n_my = jnp.maximum(jnp.minimum(cpw, total_chunks - start), 0)

    pltpu.sync_copy(z128.at[pl.ds(r0, rpt)], acc_sh.at[pl.ds(r0, rpt)])
    one = jnp.ones((16,), jnp.float32)

    def fill_body(r, carry):
        for j in range(8):
            ones_v[r, pl.ds(j * 16, 16)] = one
        return carry
    lax.fori_loop(0, CHUNK, fill_body, 0, unroll=False)
    plsc.subcore_barrier()

    def chunk_body(i, carry):
        ebase = (start + i) * CHUNK
        pltpu.sync_copy(dst.at[pl.ds(ebase, CHUNK)], di_v)
        pltpu.sync_copy(ones_v, acc_sh.at[di_v], add=True)
        return carry
    lax.fori_loop(0, n_my, chunk_body, 0, unroll=False)

    plsc.subcore_barrier()
    pltpu.sync_copy(acc_sh.at[pl.ds(r0, rpt)],
                    deg_o.at[pl.ds(cid * n_pad + r0, rpt)])


def _make_deg_kernel(n_pad, n_edges):
    mesh = plsc.VectorSubcoreMesh(core_axis_name="c", subcore_axis_name="s")
    return pl.kernel(
        functools.partial(_deg_sc_body, n_pad, n_edges),
        out_type=jax.ShapeDtypeStruct((NC * n_pad, 128), jnp.float32),
        mesh=mesh,
        scratch_types=[
            pltpu.VMEM((CHUNK,), jnp.int32),
            pltpu.VMEM((CHUNK, 128), jnp.float32),
            pltpu.VMEM_SHARED((n_pad, 128), jnp.float32),
        ],
        name="degree_sc",
    )


REPEATS = 4


def kernel(x, x_mask, edge_attr, pos, sampling_points,
           W_enc, b_enc, W_pos, b_pos, W_msg, b_msg, W_upd, b_upd,
           W_dec, b_dec, edge_index, batch):
    f32 = jnp.float32
    n = x.shape[0]
    e = edge_index.shape[1]
    H = W_enc.shape[1]

    h0 = jnp.concatenate([x, x_mask], axis=1)
    batch2 = batch[:, None]
    batch_r = batch[None, :]
    bc2 = x_mask[:, 1:2]
    src = edge_index[0]
    dst = edge_index[1]
    Wm_s = W_msg[:H]
    Wm_d = W_msg[H:2 * H]
    Wm_e = W_msg[2 * H:2 * H + 4]
    Wm_p = W_msg[2 * H + 4:]
    W1 = W_upd[:H]
    W2 = W_upd[H:2 * H]
    W3 = W_upd[2 * H:3 * H]
    W4 = W_upd[3 * H:]
    n_pad = -(-n // (8 * NS)) * (8 * NS)
    z128 = jnp.zeros((n_pad, 128), f32)

    nf = jax.ShapeDtypeStruct((n, H), f32)
    gf = jax.ShapeDtypeStruct((G, H), f32)
    h, P, oh, ohnt, crow, xbc4b, A2, B2 = pl.pallas_call(
        _pre_body,
        out_shape=(nf, nf, jax.ShapeDtypeStruct((n, G), f32),
                   jax.ShapeDtypeStruct((G, n), f32), gf, gf, nf, nf),
        name="pre_tc",
    )(h0, batch2, batch_r, bc2, pos, W_enc, b_enc[None, :], Wm_s, Wm_d, Wm_p,
      W3, W4, b_upd[None, :])

    eb = 3200
    C = pl.pallas_call(
        _c_body,
        grid=(e // eb,),
        in_specs=[pl.BlockSpec((eb, 4), lambda i: (i, 0)),
                  pl.BlockSpec((4, H), lambda i: (0, 0)),
                  pl.BlockSpec((1, H), lambda i: (0, 0))],
        out_specs=pl.BlockSpec((eb, H), lambda i: (i, 0)),
        out_shape=jax.ShapeDtypeStruct((e, H), f32),
        name="edge_const_tc",
    )(edge_attr, Wm_e, b_msg[None, :])

    edge_k = _make_edge_kernel(n_pad, e)
    degp = _make_deg_kernel(n_pad, e)(dst, z128)
    d0 = degp[:n, :1]
    d1 = degp[n_pad:n_pad + n, :1]
    upd = pl.pallas_call(
        _upd_body,
        out_shape=(nf, nf, nf, gf),
        name="update_tc",
    )

    for _ in range(REPEATS):
        aggp = edge_k(A2, B2, C, src, dst, z128)
        h, A2, B2, crow = upd(
            h, aggp[:n], aggp[n_pad:n_pad + n], d0, d1,
            oh, ohnt, crow, P, xbc4b, W1, W2, Wm_s, Wm_d, W3)

    U, nodes = pl.pallas_call(
        _epi_body,
        out_shape=(jax.ShapeDtypeStruct((sampling_points.shape[0], 4), f32),
                   jax.ShapeDtypeStruct((n, 4), f32)),
        name="epi_tc",
    )(h, sampling_points, W_dec, b_dec[None, :], W_pos, b_pos[None, :])
    return (U, nodes)


# speedup vs baseline: 6.6189x; 1.2224x over previous
import functools

import jax
import jax.numpy as jnp
from jax import lax
from jax.experimental import pallas as pl
from jax.experimental.pallas import tpu as pltpu
from jax.experimental.pallas import tpu_sc as plsc

G = 16
NC = 2
NS = 16
CHUNK = 64


def _pre_body(h0, batch2, batch_r, bc2, pos, W_enc, b_enc, Wm_s, Wm_d, Wm_p,
              W3, W4, b_upd,
              h_o, P_o, oh_o, ohnt_o, crow_o, xbc4b_o, A_o, B_o):
    f32 = jnp.float32
    h = jnp.maximum(jnp.dot(h0[...], W_enc[...],
                            preferred_element_type=f32) + b_enc[...], 0.0)
    n = h.shape[0]
    iota_cols = lax.broadcasted_iota(jnp.int32, (n, G), 1)
    oh = (batch2[...] == iota_cols).astype(f32)
    iota_rows = lax.broadcasted_iota(jnp.int32, (G, n), 0)
    oht = (batch_r[...] == iota_rows).astype(f32)
    cnt = jnp.sum(oht, axis=1, keepdims=True)
    ohnt = oht / jnp.maximum(cnt, 1.0)
    bc = (bc2[...] > 0.5).astype(f32)
    ohbct = oht * jnp.reshape(bc, (1, n))
    cnt_bc = jnp.maximum(jnp.sum(ohbct, axis=1, keepdims=True), 1.0)
    x_bc = jnp.dot(ohbct, h, preferred_element_type=f32) / cnt_bc
    xg = jnp.dot(ohnt, h, preferred_element_type=f32)
    xbc4b = jnp.dot(x_bc, W4[...], preferred_element_type=f32) + b_upd[...]
    crow = jnp.dot(xg, W3[...], preferred_element_type=f32) + xbc4b
    P = jnp.dot(pos[...], Wm_p[...], preferred_element_type=f32)
    h_o[...] = h
    P_o[...] = P
    oh_o[...] = oh
    ohnt_o[...] = ohnt
    crow_o[...] = crow
    xbc4b_o[...] = xbc4b
    A_o[...] = jnp.dot(h, Wm_s[...], preferred_element_type=f32) - P
    B_o[...] = jnp.dot(h, Wm_d[...], preferred_element_type=f32) + P


def _c_body(ea, Wm_e, b_msg, c_o):
    c_o[...] = (jnp.dot(ea[...], Wm_e[...], preferred_element_type=jnp.float32)
                + b_msg[...])


def _upd_body(h, a0, a1, d0, d1, oh, ohnt, crow, P, xbc4b,
              W1, W2, Wm_s, Wm_d, W3,
              h_o, A_o, B_o, crow_o):
    f32 = jnp.float32
    deg = jnp.maximum(d0[...] + d1[...], 1.0)
    agg = (a0[...] + a1[...]) / deg
    u = jnp.maximum(
        jnp.dot(h[...], W1[...], preferred_element_type=f32)
        + jnp.dot(agg, W2[...], preferred_element_type=f32)
        + jnp.dot(oh[...], crow[...], preferred_element_type=f32), 0.0)
    h2 = h[...] + u
    xg = jnp.dot(ohnt[...], h2, preferred_element_type=f32)
    h_o[...] = h2
    A_o[...] = jnp.dot(h2, Wm_s[...], preferred_element_type=f32) - P[...]
    B_o[...] = jnp.dot(h2, Wm_d[...], preferred_element_type=f32) + P[...]
    crow_o[...] = jnp.dot(xg, W3[...], preferred_element_type=f32) + xbc4b[...]


def _epi_body(h, sp, W_dec, b_dec, W_pos, b_pos, u_o, nodes_o):
    f32 = jnp.float32
    nodes_o[...] = (jnp.dot(h[...], W_dec[...], preferred_element_type=f32)
                    + b_dec[...])
    es = jnp.maximum(jnp.dot(sp[...], W_pos[...],
                             preferred_element_type=f32) + b_pos[...], 0.0)
    u_o[...] = jnp.dot(es, W_dec[...], preferred_element_type=f32) + b_dec[...]


def _chunk_ranges(n_edges):
    total_chunks = n_edges // CHUNK
    cpw = -(-total_chunks // (NC * NS))
    return total_chunks, cpw


_HI = -65536


def _edge_sc_body(n_pad, n_edges,
                  A2, B2, C, src, dst, z128,
                  agg_o,
                  si_v, di_v, a_v, b_v, c_v,
                  acc_sh, sem_a, sem_b, sem_c, sem_s, sem_si, sem_di):
    cid = lax.axis_index("c")
    sid = lax.axis_index("s")
    wid = sid * NC + cid
    rpt = n_pad // NS
    r0 = sid * rpt

    total_chunks, cpw = _chunk_ranges(n_edges)
    start = wid * cpw
    n_my = jnp.maximum(jnp.minimum(cpw, total_chunks - start), 0)

    pltpu.sync_copy(z128.at[pl.ds(r0, rpt)], acc_sh.at[pl.ds(r0, rpt)])
    plsc.subcore_barrier()

    def idx_fetch(ci):
        s4 = lax.rem(ci, 4)
        ebase = (start + ci) * CHUNK
        pltpu.async_copy(src.at[pl.ds(ebase, CHUNK)], si_v.at[s4],
                         sem_si.at[s4])
        pltpu.async_copy(dst.at[pl.ds(ebase, CHUNK)], di_v.at[s4],
                         sem_di.at[s4])

    def idx_wait(ci):
        s4 = lax.rem(ci, 4)
        pltpu.make_async_copy(src.at[pl.ds(0, CHUNK)], si_v.at[s4],
                              sem_si.at[s4]).wait()
        pltpu.make_async_copy(dst.at[pl.ds(0, CHUNK)], di_v.at[s4],
                              sem_di.at[s4]).wait()

    def stage(slot, ci):
        ebase = (start + ci) * CHUNK
        s4 = lax.rem(ci, 4)
        pltpu.async_copy(A2.at[si_v.at[s4]], a_v.at[slot], sem_a.at[slot])
        pltpu.async_copy(B2.at[di_v.at[s4]], b_v.at[slot], sem_b.at[slot])
        pltpu.async_copy(C.at[pl.ds(ebase, CHUNK)], c_v.at[slot],
                         sem_c.at[slot])

    @pl.when(n_my > 0)
    def _():
        idx_fetch(0)
        idx_wait(0)
        stage(0, 0)

    @pl.when(n_my > 1)
    def _():
        idx_fetch(1)

    def chunk_body(i, carry):
        p = lax.rem(i, 2)
        q = lax.rem(i + 1, 2)
        ebase = (start + i) * CHUNK

        @pl.when(i >= 1)
        def _():
            pltpu.make_async_copy(c_v.at[q], acc_sh.at[di_v.at[lax.rem(i - 1, 4)]],
                                  sem_s.at[q]).wait()

        @pl.when(i + 2 < n_my)
        def _():
            idx_fetch(i + 2)

        @pl.when(i + 1 < n_my)
        def _():
            idx_wait(i + 1)
            stage(q, i + 1)

        pltpu.make_async_copy(A2.at[si_v.at[lax.rem(i, 4)]], a_v.at[p],
                              sem_a.at[p]).wait()
        pltpu.make_async_copy(B2.at[di_v.at[lax.rem(i, 4)]], b_v.at[p],
                              sem_b.at[p]).wait()
        pltpu.make_async_copy(C.at[pl.ds(ebase, CHUNK)], c_v.at[p],
                              sem_c.at[p]).wait()

        @plsc.parallel_loop(0, CHUNK, unroll=2)
        def row_body(r):
            for j in range(8):
                s = pl.ds(j * 16, 16)
                c_v[p, r, s] = jnp.maximum(
                    a_v[p, r, s] + b_v[p, r, s] + c_v[p, r, s], 0.0)

        pltpu.async_copy(c_v.at[p], acc_sh.at[di_v.at[lax.rem(i, 4)]],
                         sem_s.at[p], add=True)
        return carry
    lax.fori_loop(0, n_my, chunk_body, 0, unroll=False)

    @pl.when(n_my >= 1)
    def _():
        pf = lax.rem(n_my - 1, 2)
        pltpu.make_async_copy(c_v.at[pf],
                              acc_sh.at[di_v.at[lax.rem(n_my - 1, 4)]],
                              sem_s.at[pf]).wait()

    plsc.subcore_barrier()
    pltpu.sync_copy(acc_sh.at[pl.ds(r0, rpt)],
                    agg_o.at[pl.ds(cid * n_pad + r0, rpt)])


def _make_edge_kernel(n_pad, n_edges):
    mesh = plsc.VectorSubcoreMesh(core_axis_name="c", subcore_axis_name="s")
    return pl.kernel(
        functools.partial(_edge_sc_body, n_pad, n_edges),
        out_type=jax.ShapeDtypeStruct((NC * n_pad, 128), jnp.float32),
        mesh=mesh,
        scratch_types=[
            pltpu.VMEM((4, CHUNK), jnp.int32),
            pltpu.VMEM((4, CHUNK), jnp.int32),
            pltpu.VMEM((2, CHUNK, 128), jnp.float32),
            pltpu.VMEM((2, CHUNK, 128), jnp.float32),
            pltpu.VMEM((2, CHUNK, 128), jnp.float32),
            pltpu.VMEM_SHARED((n_pad, 128), jnp.float32),
            pltpu.SemaphoreType.DMA((2,)),
            pltpu.SemaphoreType.DMA((2,)),
            pltpu.SemaphoreType.DMA((2,)),
            pltpu.SemaphoreType.DMA((2,)),
            pltpu.SemaphoreType.DMA((4,)),
            pltpu.SemaphoreType.DMA((4,)),
        ],
        name="edge_messages_sc",
    )


def _deg_sc_body(n_pad, n_edges,
                 dst, z128,
                 deg_o,
                 di_v, ones_v, acc_sh):
    cid = lax.axis_index("c")
    sid = lax.axis_index("s")
    wid = sid * NC + cid
    rpt = n_pad // NS
    r0 = sid * rpt

    total_chunks, cpw = _chunk_ranges(n_edges)
    start = wid * cpw
    n_my = jnp.maximum(jnp.minimum(cpw, total_chunks - start), 0)

    pltpu.sync_copy(z128.at[pl.ds(r0, rpt)], acc_sh.at[pl.ds(r0, rpt)])
    one = jnp.ones((16,), jnp.float32)

    def fill_body(r, carry):
        for j in range(8):
            ones_v[r, pl.ds(j * 16, 16)] = one
        return carry
    lax.fori_loop(0, CHUNK, fill_body, 0, unroll=False)
    plsc.subcore_barrier()

    def chunk_body(i, carry):
        ebase = (start + i) * CHUNK
        pltpu.sync_copy(dst.at[pl.ds(ebase, CHUNK)], di_v)
        pltpu.sync_copy(ones_v, acc_sh.at[di_v], add=True)
        return carry
    lax.fori_loop(0, n_my, chunk_body, 0, unroll=False)

    plsc.subcore_barrier()
    pltpu.sync_copy(acc_sh.at[pl.ds(r0, rpt)],
                    deg_o.at[pl.ds(cid * n_pad + r0, rpt)])


def _make_deg_kernel(n_pad, n_edges):
    mesh = plsc.VectorSubcoreMesh(core_axis_name="c", subcore_axis_name="s")
    return pl.kernel(
        functools.partial(_deg_sc_body, n_pad, n_edges),
        out_type=jax.ShapeDtypeStruct((NC * n_pad, 128), jnp.float32),
        mesh=mesh,
        scratch_types=[
            pltpu.VMEM((CHUNK,), jnp.int32),
            pltpu.VMEM((CHUNK, 128), jnp.float32),
            pltpu.VMEM_SHARED((n_pad, 128), jnp.float32),
        ],
        name="degree_sc",
    )


REPEATS = 4


def kernel(x, x_mask, edge_attr, pos, sampling_points,
           W_enc, b_enc, W_pos, b_pos, W_msg, b_msg, W_upd, b_upd,
           W_dec, b_dec, edge_index, batch):
    f32 = jnp.float32
    n = x.shape[0]
    e = edge_index.shape[1]
    H = W_enc.shape[1]

    h0 = jnp.concatenate([x, x_mask], axis=1)
    batch2 = batch[:, None]
    batch_r = batch[None, :]
    bc2 = x_mask[:, 1:2]
    src = edge_index[0]
    dst = edge_index[1]
    Wm_s = W_msg[:H]
    Wm_d = W_msg[H:2 * H]
    Wm_e = W_msg[2 * H:2 * H + 4]
    Wm_p = W_msg[2 * H + 4:]
    W1 = W_upd[:H]
    W2 = W_upd[H:2 * H]
    W3 = W_upd[2 * H:3 * H]
    W4 = W_upd[3 * H:]
    n_pad = -(-n // (8 * NS)) * (8 * NS)
    z128 = jnp.zeros((n_pad, 128), f32)

    nf = jax.ShapeDtypeStruct((n, H), f32)
    gf = jax.ShapeDtypeStruct((G, H), f32)
    h, P, oh, ohnt, crow, xbc4b, A2, B2 = pl.pallas_call(
        _pre_body,
        out_shape=(nf, nf, jax.ShapeDtypeStruct((n, G), f32),
                   jax.ShapeDtypeStruct((G, n), f32), gf, gf, nf, nf),
        name="pre_tc",
    )(h0, batch2, batch_r, bc2, pos, W_enc, b_enc[None, :], Wm_s, Wm_d, Wm_p,
      W3, W4, b_upd[None, :])

    eb = 3200
    C = pl.pallas_call(
        _c_body,
        grid=(e // eb,),
        in_specs=[pl.BlockSpec((eb, 4), lambda i: (i, 0)),
                  pl.BlockSpec((4, H), lambda i: (0, 0)),
                  pl.BlockSpec((1, H), lambda i: (0, 0))],
        out_specs=pl.BlockSpec((eb, H), lambda i: (i, 0)),
        out_shape=jax.ShapeDtypeStruct((e, H), f32),
        name="edge_const_tc",
    )(edge_attr, Wm_e, b_msg[None, :])

    edge_k = _make_edge_kernel(n_pad, e)
    degp = _make_deg_kernel(n_pad, e)(dst, z128)
    d0 = degp[:n, :1]
    d1 = degp[n_pad:n_pad + n, :1]
    upd = pl.pallas_call(
        _upd_body,
        out_shape=(nf, nf, nf, gf),
        name="update_tc",
    )

    for _ in range(REPEATS):
        aggp = edge_k(A2, B2, C, src, dst, z128)
        h, A2, B2, crow = upd(
            h, aggp[:n], aggp[n_pad:n_pad + n], d0, d1,
            oh, ohnt, crow, P, xbc4b, W1, W2, Wm_s, Wm_d, W3)

    U, nodes = pl.pallas_call(
        _epi_body,
        out_shape=(jax.ShapeDtypeStruct((sampling_points.shape[0], 4), f32),
                   jax.ShapeDtypeStruct((n, 4), f32)),
        name="epi_tc",
    )(h, sampling_points, W_dec, b_dec[None, :], W_pos, b_pos[None, :])
    return (U, nodes)


# speedup vs baseline: 6.7627x; 1.0217x over previous
import functools

import jax
import jax.numpy as jnp
from jax import lax
from jax.experimental import pallas as pl
from jax.experimental.pallas import tpu as pltpu
from jax.experimental.pallas import tpu_sc as plsc

G = 16
NC = 2
NS = 16
CHUNK = 64


def _pre_body(h0, batch2, batch_r, bc2, pos, W_enc, b_enc, Wm_s, Wm_d, Wm_p,
              W3, W4, b_upd,
              h_o, P_o, oh_o, ohnt_o, crow_o, xbc4b_o, A_o, B_o):
    f32 = jnp.float32
    h = jnp.maximum(jnp.dot(h0[...], W_enc[...],
                            preferred_element_type=f32) + b_enc[...], 0.0)
    n = h.shape[0]
    iota_cols = lax.broadcasted_iota(jnp.int32, (n, G), 1)
    oh = (batch2[...] == iota_cols).astype(f32)
    iota_rows = lax.broadcasted_iota(jnp.int32, (G, n), 0)
    oht = (batch_r[...] == iota_rows).astype(f32)
    cnt = jnp.sum(oht, axis=1, keepdims=True)
    ohnt = oht / jnp.maximum(cnt, 1.0)
    bc = (bc2[...] > 0.5).astype(f32)
    ohbct = oht * jnp.reshape(bc, (1, n))
    cnt_bc = jnp.maximum(jnp.sum(ohbct, axis=1, keepdims=True), 1.0)
    x_bc = jnp.dot(ohbct, h, preferred_element_type=f32) / cnt_bc
    xg = jnp.dot(ohnt, h, preferred_element_type=f32)
    xbc4b = jnp.dot(x_bc, W4[...], preferred_element_type=f32) + b_upd[...]
    crow = jnp.dot(xg, W3[...], preferred_element_type=f32) + xbc4b
    P = jnp.dot(pos[...], Wm_p[...], preferred_element_type=f32)
    h_o[...] = h
    P_o[...] = P
    oh_o[...] = oh
    ohnt_o[...] = ohnt
    crow_o[...] = crow
    xbc4b_o[...] = xbc4b
    A_o[...] = jnp.dot(h, Wm_s[...], preferred_element_type=f32) - P
    B_o[...] = jnp.dot(h, Wm_d[...], preferred_element_type=f32) + P


def _c_body(ea, Wm_e, b_msg, c_o):
    c_o[...] = (jnp.dot(ea[...], Wm_e[...], preferred_element_type=jnp.float32)
                + b_msg[...])


def _upd_body(h, a0, a1, d0, d1, oh, ohnt, crow, P, xbc4b,
              W1, W2, Wm_s, Wm_d, W3,
              h_o, A_o, B_o, crow_o):
    f32 = jnp.float32
    deg = jnp.maximum(d0[...] + d1[...], 1.0)
    agg = (a0[...] + a1[...]) / deg
    u = jnp.maximum(
        jnp.dot(h[...], W1[...], preferred_element_type=f32)
        + jnp.dot(agg, W2[...], preferred_element_type=f32)
        + jnp.dot(oh[...], crow[...], preferred_element_type=f32), 0.0)
    h2 = h[...] + u
    xg = jnp.dot(ohnt[...], h2, preferred_element_type=f32)
    h_o[...] = h2
    A_o[...] = jnp.dot(h2, Wm_s[...], preferred_element_type=f32) - P[...]
    B_o[...] = jnp.dot(h2, Wm_d[...], preferred_element_type=f32) + P[...]
    crow_o[...] = jnp.dot(xg, W3[...], preferred_element_type=f32) + xbc4b[...]


def _epi_body(h, sp, W_dec, b_dec, W_pos, b_pos, u_o, nodes_o):
    f32 = jnp.float32
    nodes_o[...] = (jnp.dot(h[...], W_dec[...], preferred_element_type=f32)
                    + b_dec[...])
    es = jnp.maximum(jnp.dot(sp[...], W_pos[...],
                             preferred_element_type=f32) + b_pos[...], 0.0)
    u_o[...] = jnp.dot(es, W_dec[...], preferred_element_type=f32) + b_dec[...]


def _chunk_ranges(n_edges):
    total_chunks = n_edges // CHUNK
    cpw = -(-total_chunks // (NC * NS))
    return total_chunks, cpw


_HI = -65536


def _edge_sc_body(n_pad, n_edges,
                  A2, B2, C, src, dst, z128,
                  agg_o,
                  si_v, di_v, a_v, b_v, c_v,
                  acc_sh, sem_a, sem_b, sem_c, sem_s, sem_si, sem_di):
    cid = lax.axis_index("c")
    sid = lax.axis_index("s")
    wid = sid * NC + cid
    rpt = n_pad // NS
    r0 = sid * rpt

    total_chunks, cpw = _chunk_ranges(n_edges)
    start = wid * cpw
    n_my = jnp.maximum(jnp.minimum(cpw, total_chunks - start), 0)

    pltpu.sync_copy(z128.at[pl.ds(r0, rpt)], acc_sh.at[pl.ds(r0, rpt)])
    plsc.subcore_barrier()

    def idx_fetch(ci):
        s4 = lax.rem(ci, 4)
        ebase = (start + ci) * CHUNK
        pltpu.async_copy(src.at[pl.ds(ebase, CHUNK)], si_v.at[s4],
                         sem_si.at[s4])
        pltpu.async_copy(dst.at[pl.ds(ebase, CHUNK)], di_v.at[s4],
                         sem_di.at[s4])

    def idx_wait(ci):
        s4 = lax.rem(ci, 4)
        pltpu.make_async_copy(src.at[pl.ds(0, CHUNK)], si_v.at[s4],
                              sem_si.at[s4]).wait()
        pltpu.make_async_copy(dst.at[pl.ds(0, CHUNK)], di_v.at[s4],
                              sem_di.at[s4]).wait()

    def stage(slot, ci):
        ebase = (start + ci) * CHUNK
        s4 = lax.rem(ci, 4)
        pltpu.async_copy(A2.at[si_v.at[s4]], a_v.at[slot], sem_a.at[slot])
        pltpu.async_copy(B2.at[di_v.at[s4]], b_v.at[slot], sem_b.at[slot])
        pltpu.async_copy(C.at[pl.ds(ebase, CHUNK)], c_v.at[slot],
                         sem_c.at[slot])

    @pl.when(n_my > 0)
    def _():
        idx_fetch(0)
        idx_wait(0)
        stage(0, 0)

    @pl.when(n_my > 1)
    def _():
        idx_fetch(1)

    def chunk_body(i, carry):
        p = lax.rem(i, 2)
        q = lax.rem(i + 1, 2)
        ebase = (start + i) * CHUNK

        @pl.when(i >= 1)
        def _():
            pltpu.make_async_copy(c_v.at[q], acc_sh.at[di_v.at[lax.rem(i - 1, 4)]],
                                  sem_s.at[q]).wait()

        @pl.when(i + 2 < n_my)
        def _():
            idx_fetch(i + 2)

        @pl.when(i + 1 < n_my)
        def _():
            idx_wait(i + 1)
            stage(q, i + 1)

        pltpu.make_async_copy(A2.at[si_v.at[lax.rem(i, 4)]], a_v.at[p],
                              sem_a.at[p]).wait()
        pltpu.make_async_copy(B2.at[di_v.at[lax.rem(i, 4)]], b_v.at[p],
                              sem_b.at[p]).wait()
        pltpu.make_async_copy(C.at[pl.ds(ebase, CHUNK)], c_v.at[p],
                              sem_c.at[p]).wait()

        @plsc.parallel_loop(0, CHUNK, unroll=4)
        def row_body(r):
            for j in range(8):
                s = pl.ds(j * 16, 16)
                c_v[p, r, s] = jnp.maximum(
                    a_v[p, r, s] + b_v[p, r, s] + c_v[p, r, s], 0.0)

        pltpu.async_copy(c_v.at[p], acc_sh.at[di_v.at[lax.rem(i, 4)]],
                         sem_s.at[p], add=True)
        return carry
    lax.fori_loop(0, n_my, chunk_body, 0, unroll=False)

    @pl.when(n_my >= 1)
    def _():
        pf = lax.rem(n_my - 1, 2)
        pltpu.make_async_copy(c_v.at[pf],
                              acc_sh.at[di_v.at[lax.rem(n_my - 1, 4)]],
                              sem_s.at[pf]).wait()

    plsc.subcore_barrier()
    pltpu.sync_copy(acc_sh.at[pl.ds(r0, rpt)],
                    agg_o.at[pl.ds(cid * n_pad + r0, rpt)])


def _make_edge_kernel(n_pad, n_edges):
    mesh = plsc.VectorSubcoreMesh(core_axis_name="c", subcore_axis_name="s")
    return pl.kernel(
        functools.partial(_edge_sc_body, n_pad, n_edges),
        out_type=jax.ShapeDtypeStruct((NC * n_pad, 128), jnp.float32),
        mesh=mesh,
        scratch_types=[
            pltpu.VMEM((4, CHUNK), jnp.int32),
            pltpu.VMEM((4, CHUNK), jnp.int32),
            pltpu.VMEM((2, CHUNK, 128), jnp.float32),
            pltpu.VMEM((2, CHUNK, 128), jnp.float32),
            pltpu.VMEM((2, CHUNK, 128), jnp.float32),
            pltpu.VMEM_SHARED((n_pad, 128), jnp.float32),
            pltpu.SemaphoreType.DMA((2,)),
            pltpu.SemaphoreType.DMA((2,)),
            pltpu.SemaphoreType.DMA((2,)),
            pltpu.SemaphoreType.DMA((2,)),
            pltpu.SemaphoreType.DMA((4,)),
            pltpu.SemaphoreType.DMA((4,)),
        ],
        name="edge_messages_sc",
    )


def _deg_sc_body(n_pad, n_edges,
                 dst, z128,
                 deg_o,
                 di_v, ones_v, acc_sh):
    cid = lax.axis_index("c")
    sid = lax.axis_index("s")
    wid = sid * NC + cid
    rpt = n_pad // NS
    r0 = sid * rpt

    total_chunks, cpw = _chunk_ranges(n_edges)
    start = wid * cpw
    n_my = jnp.maximum(jnp.minimum(cpw, total_chunks - start), 0)

    pltpu.sync_copy(z128.at[pl.ds(r0, rpt)], acc_sh.at[pl.ds(r0, rpt)])
    one = jnp.ones((16,), jnp.float32)

    def fill_body(r, carry):
        for j in range(8):
            ones_v[r, pl.ds(j * 16, 16)] = one
        return carry
    lax.fori_loop(0, CHUNK, fill_body, 0, unroll=False)
    plsc.subcore_barrier()

    def chunk_body(i, carry):
        ebase = (start + i) * CHUNK
        pltpu.sync_copy(dst.at[pl.ds(ebase, CHUNK)], di_v)
        pltpu.sync_copy(ones_v, acc_sh.at[di_v], add=True)
        return carry
    lax.fori_loop(0, n_my, chunk_body, 0, unroll=False)

    plsc.subcore_barrier()
    pltpu.sync_copy(acc_sh.at[pl.ds(r0, rpt)],
                    deg_o.at[pl.ds(cid * n_pad + r0, rpt)])


def _make_deg_kernel(n_pad, n_edges):
    mesh = plsc.VectorSubcoreMesh(core_axis_name="c", subcore_axis_name="s")
    return pl.kernel(
        functools.partial(_deg_sc_body, n_pad, n_edges),
        out_type=jax.ShapeDtypeStruct((NC * n_pad, 128), jnp.float32),
        mesh=mesh,
        scratch_types=[
            pltpu.VMEM((CHUNK,), jnp.int32),
            pltpu.VMEM((CHUNK, 128), jnp.float32),
            pltpu.VMEM_SHARED((n_pad, 128), jnp.float32),
        ],
        name="degree_sc",
    )


REPEATS = 4


def kernel(x, x_mask, edge_attr, pos, sampling_points,
           W_enc, b_enc, W_pos, b_pos, W_msg, b_msg, W_upd, b_upd,
           W_dec, b_dec, edge_index, batch):
    f32 = jnp.float32
    n = x.shape[0]
    e = edge_index.shape[1]
    H = W_enc.shape[1]

    h0 = jnp.concatenate([x, x_mask], axis=1)
    batch2 = batch[:, None]
    batch_r = batch[None, :]
    bc2 = x_mask[:, 1:2]
    src = edge_index[0]
    dst = edge_index[1]
    Wm_s = W_msg[:H]
    Wm_d = W_msg[H:2 * H]
    Wm_e = W_msg[2 * H:2 * H + 4]
    Wm_p = W_msg[2 * H + 4:]
    W1 = W_upd[:H]
    W2 = W_upd[H:2 * H]
    W3 = W_upd[2 * H:3 * H]
    W4 = W_upd[3 * H:]
    n_pad = -(-n // (8 * NS)) * (8 * NS)
    z128 = jnp.zeros((n_pad, 128), f32)

    nf = jax.ShapeDtypeStruct((n, H), f32)
    gf = jax.ShapeDtypeStruct((G, H), f32)
    h, P, oh, ohnt, crow, xbc4b, A2, B2 = pl.pallas_call(
        _pre_body,
        out_shape=(nf, nf, jax.ShapeDtypeStruct((n, G), f32),
                   jax.ShapeDtypeStruct((G, n), f32), gf, gf, nf, nf),
        name="pre_tc",
    )(h0, batch2, batch_r, bc2, pos, W_enc, b_enc[None, :], Wm_s, Wm_d, Wm_p,
      W3, W4, b_upd[None, :])

    eb = 3200
    C = pl.pallas_call(
        _c_body,
        grid=(e // eb,),
        in_specs=[pl.BlockSpec((eb, 4), lambda i: (i, 0)),
                  pl.BlockSpec((4, H), lambda i: (0, 0)),
                  pl.BlockSpec((1, H), lambda i: (0, 0))],
        out_specs=pl.BlockSpec((eb, H), lambda i: (i, 0)),
        out_shape=jax.ShapeDtypeStruct((e, H), f32),
        name="edge_const_tc",
    )(edge_attr, Wm_e, b_msg[None, :])

    edge_k = _make_edge_kernel(n_pad, e)
    degp = _make_deg_kernel(n_pad, e)(dst, z128)
    d0 = degp[:n, :1]
    d1 = degp[n_pad:n_pad + n, :1]
    upd = pl.pallas_call(
        _upd_body,
        out_shape=(nf, nf, nf, gf),
        name="update_tc",
    )

    for _ in range(REPEATS):
        aggp = edge_k(A2, B2, C, src, dst, z128)
        h, A2, B2, crow = upd(
            h, aggp[:n], aggp[n_pad:n_pad + n], d0, d1,
            oh, ohnt, crow, P, xbc4b, W1, W2, Wm_s, Wm_d, W3)

    U, nodes = pl.pallas_call(
        _epi_body,
        out_shape=(jax.ShapeDtypeStruct((sampling_points.shape[0], 4), f32),
                   jax.ShapeDtypeStruct((n, 4), f32)),
        name="epi_tc",
    )(h, sampling_points, W_dec, b_dec[None, :], W_pos, b_pos[None, :])
    return (U, nodes)


# speedup vs baseline: 7.0635x; 1.0445x over previous
import functools

import jax
import jax.numpy as jnp
from jax import lax
from jax.experimental import pallas as pl
from jax.experimental.pallas import tpu as pltpu
from jax.experimental.pallas import tpu_sc as plsc

G = 16
NC = 2
NS = 16
CHUNK = 64


def _pre_body(h0, batch2, batch_r, bc2, pos, W_enc, b_enc, Wm_s, Wm_d, Wm_p,
              W3, W4, b_upd,
              h_o, P_o, oh_o, ohnt_o, crow_o, xbc4b_o, A_o, B_o):
    f32 = jnp.float32
    h = jnp.maximum(jnp.dot(h0[...], W_enc[...],
                            preferred_element_type=f32) + b_enc[...], 0.0)
    n = h.shape[0]
    iota_cols = lax.broadcasted_iota(jnp.int32, (n, G), 1)
    oh = (batch2[...] == iota_cols).astype(f32)
    iota_rows = lax.broadcasted_iota(jnp.int32, (G, n), 0)
    oht = (batch_r[...] == iota_rows).astype(f32)
    cnt = jnp.sum(oht, axis=1, keepdims=True)
    ohnt = oht / jnp.maximum(cnt, 1.0)
    bc = (bc2[...] > 0.5).astype(f32)
    ohbct = oht * jnp.reshape(bc, (1, n))
    cnt_bc = jnp.maximum(jnp.sum(ohbct, axis=1, keepdims=True), 1.0)
    x_bc = jnp.dot(ohbct, h, preferred_element_type=f32) / cnt_bc
    xg = jnp.dot(ohnt, h, preferred_element_type=f32)
    xbc4b = jnp.dot(x_bc, W4[...], preferred_element_type=f32) + b_upd[...]
    crow = jnp.dot(xg, W3[...], preferred_element_type=f32) + xbc4b
    P = jnp.dot(pos[...], Wm_p[...], preferred_element_type=f32)
    h_o[...] = h
    P_o[...] = P
    oh_o[...] = oh
    ohnt_o[...] = ohnt
    crow_o[...] = crow
    xbc4b_o[...] = xbc4b
    A_o[...] = jnp.dot(h, Wm_s[...], preferred_element_type=f32) - P
    B_o[...] = jnp.dot(h, Wm_d[...], preferred_element_type=f32) + P


def _c_body(ea, Wm_e, b_msg, c_o):
    c_o[...] = (jnp.dot(ea[...], Wm_e[...], preferred_element_type=jnp.float32)
                + b_msg[...])


def _upd_body(h, a0, a1, d0, d1, oh, ohnt, crow, P, xbc4b,
              W1, W2, Wm_s, Wm_d, W3,
              h_o, A_o, B_o, crow_o):
    f32 = jnp.float32
    deg = jnp.maximum(d0[...] + d1[...], 1.0)
    agg = (a0[...] + a1[...]) / deg
    u = jnp.maximum(
        jnp.dot(h[...], W1[...], preferred_element_type=f32)
        + jnp.dot(agg, W2[...], preferred_element_type=f32)
        + jnp.dot(oh[...], crow[...], preferred_element_type=f32), 0.0)
    h2 = h[...] + u
    xg = jnp.dot(ohnt[...], h2, preferred_element_type=f32)
    h_o[...] = h2
    A_o[...] = jnp.dot(h2, Wm_s[...], preferred_element_type=f32) - P[...]
    B_o[...] = jnp.dot(h2, Wm_d[...], preferred_element_type=f32) + P[...]
    crow_o[...] = jnp.dot(xg, W3[...], preferred_element_type=f32) + xbc4b[...]


def _epi_body(h, sp, W_dec, b_dec, W_pos, b_pos, u_o, nodes_o):
    f32 = jnp.float32
    nodes_o[...] = (jnp.dot(h[...], W_dec[...], preferred_element_type=f32)
                    + b_dec[...])
    es = jnp.maximum(jnp.dot(sp[...], W_pos[...],
                             preferred_element_type=f32) + b_pos[...], 0.0)
    u_o[...] = jnp.dot(es, W_dec[...], preferred_element_type=f32) + b_dec[...]


def _chunk_ranges(n_edges):
    total_chunks = n_edges // CHUNK
    cpw = -(-total_chunks // (NC * NS))
    return total_chunks, cpw


_HI = -65536


def _edge_sc_body(n_pad, n_edges,
                  A2, B2, C, src, dst, z128,
                  agg_o,
                  si_v, di_v, a_v, b_v, c_v,
                  acc_sh, sem_a, sem_b, sem_c, sem_s, sem_si, sem_di):
    cid = lax.axis_index("c")
    sid = lax.axis_index("s")
    wid = sid * NC + cid
    rpt = n_pad // NS
    r0 = sid * rpt

    total_chunks, cpw = _chunk_ranges(n_edges)
    start = wid * cpw
    n_my = jnp.maximum(jnp.minimum(cpw, total_chunks - start), 0)

    pltpu.sync_copy(z128.at[pl.ds(r0, rpt)], acc_sh.at[pl.ds(r0, rpt)])
    plsc.subcore_barrier()

    def idx_fetch(ci):
        s4 = lax.rem(ci, 4)
        ebase = (start + ci) * CHUNK
        pltpu.async_copy(src.at[pl.ds(ebase, CHUNK)], si_v.at[s4],
                         sem_si.at[s4])
        pltpu.async_copy(dst.at[pl.ds(ebase, CHUNK)], di_v.at[s4],
                         sem_di.at[s4])

    def idx_wait(ci):
        s4 = lax.rem(ci, 4)
        pltpu.make_async_copy(src.at[pl.ds(0, CHUNK)], si_v.at[s4],
                              sem_si.at[s4]).wait()
        pltpu.make_async_copy(dst.at[pl.ds(0, CHUNK)], di_v.at[s4],
                              sem_di.at[s4]).wait()

    def stage(slot, ci):
        ebase = (start + ci) * CHUNK
        s4 = lax.rem(ci, 4)
        pltpu.async_copy(A2.at[si_v.at[s4]], a_v.at[slot], sem_a.at[slot])
        pltpu.async_copy(B2.at[di_v.at[s4]], b_v.at[slot], sem_b.at[slot])
        pltpu.async_copy(C.at[pl.ds(ebase, CHUNK)], c_v.at[slot],
                         sem_c.at[slot])

    @pl.when(n_my > 0)
    def _():
        idx_fetch(0)
        idx_wait(0)
        stage(0, 0)

    @pl.when(n_my > 1)
    def _():
        idx_fetch(1)

    def chunk_body(i, carry):
        p = lax.rem(i, 2)
        q = lax.rem(i + 1, 2)
        ebase = (start + i) * CHUNK

        @pl.when(i >= 1)
        def _():
            pltpu.make_async_copy(c_v.at[q], acc_sh.at[di_v.at[lax.rem(i - 1, 4)]],
                                  sem_s.at[q]).wait()

        @pl.when(i + 2 < n_my)
        def _():
            idx_fetch(i + 2)

        @pl.when(i + 1 < n_my)
        def _():
            idx_wait(i + 1)
            stage(q, i + 1)

        pltpu.make_async_copy(A2.at[si_v.at[lax.rem(i, 4)]], a_v.at[p],
                              sem_a.at[p]).wait()
        pltpu.make_async_copy(B2.at[di_v.at[lax.rem(i, 4)]], b_v.at[p],
                              sem_b.at[p]).wait()
        pltpu.make_async_copy(C.at[pl.ds(ebase, CHUNK)], c_v.at[p],
                              sem_c.at[p]).wait()

        @plsc.parallel_loop(0, CHUNK, unroll=4)
        def row_body(r):
            for j in range(8):
                s = pl.ds(j * 16, 16)
                c_v[p, r, s] = jnp.maximum(
                    a_v[p, r, s] + b_v[p, r, s] + c_v[p, r, s], 0.0)

        pltpu.async_copy(c_v.at[p], acc_sh.at[di_v.at[lax.rem(i, 4)]],
                         sem_s.at[p], add=True)
        return carry
    lax.fori_loop(0, n_my, chunk_body, 0, unroll=False)

    @pl.when(n_my >= 1)
    def _():
        pf = lax.rem(n_my - 1, 2)
        pltpu.make_async_copy(c_v.at[pf],
                              acc_sh.at[di_v.at[lax.rem(n_my - 1, 4)]],
                              sem_s.at[pf]).wait()

    plsc.subcore_barrier()
    pltpu.sync_copy(acc_sh.at[pl.ds(r0, rpt)],
                    agg_o.at[pl.ds(cid * n_pad + r0, rpt)])


def _make_edge_kernel(n_pad, n_edges):
    mesh = plsc.VectorSubcoreMesh(core_axis_name="c", subcore_axis_name="s")
    return pl.kernel(
        functools.partial(_edge_sc_body, n_pad, n_edges),
        out_type=jax.ShapeDtypeStruct((NC * n_pad, 128), jnp.float32),
        mesh=mesh,
        scratch_types=[
            pltpu.VMEM((4, CHUNK), jnp.int32),
            pltpu.VMEM((4, CHUNK), jnp.int32),
            pltpu.VMEM((2, CHUNK, 128), jnp.float32),
            pltpu.VMEM((2, CHUNK, 128), jnp.float32),
            pltpu.VMEM((2, CHUNK, 128), jnp.float32),
            pltpu.VMEM_SHARED((n_pad, 128), jnp.float32),
            pltpu.SemaphoreType.DMA((2,)),
            pltpu.SemaphoreType.DMA((2,)),
            pltpu.SemaphoreType.DMA((2,)),
            pltpu.SemaphoreType.DMA((2,)),
            pltpu.SemaphoreType.DMA((4,)),
            pltpu.SemaphoreType.DMA((4,)),
        ],
        name="edge_messages_sc",
    )


def _deg_sc_body(n_pad, n_edges,
                 dst, z128,
                 deg_o,
                 di_v, ones_v, acc_sh, sem_s, sem_di):
    cid = lax.axis_index("c")
    sid = lax.axis_index("s")
    wid = sid * NC + cid
    rpt = n_pad // NS
    r0 = sid * rpt

    total_chunks, cpw = _chunk_ranges(n_edges)
    start = wid * cpw
    n_my = jnp.maximum(jnp.minimum(cpw, total_chunks - start), 0)

    pltpu.sync_copy(z128.at[pl.ds(r0, rpt)], acc_sh.at[pl.ds(r0, rpt)])
    one = jnp.ones((16,), jnp.float32)

    def fill_body(r, carry):
        for j in range(8):
            ones_v[r, pl.ds(j * 16, 16)] = one
        return carry
    lax.fori_loop(0, CHUNK, fill_body, 0, unroll=False)

    def idx_fetch(ci):
        s4 = lax.rem(ci, 4)
        pltpu.async_copy(dst.at[pl.ds((start + ci) * CHUNK, CHUNK)],
                         di_v.at[s4], sem_di.at[s4])

    @pl.when(n_my > 0)
    def _():
        idx_fetch(0)

    @pl.when(n_my > 1)
    def _():
        idx_fetch(1)
    plsc.subcore_barrier()

    def chunk_body(i, carry):
        p = lax.rem(i, 2)
        s4 = lax.rem(i, 4)

        @pl.when(i >= 2)
        def _():
            pltpu.make_async_copy(ones_v, acc_sh.at[di_v.at[lax.rem(i - 2, 4)]],
                                  sem_s.at[p]).wait()

        @pl.when(i + 2 < n_my)
        def _():
            idx_fetch(i + 2)

        pltpu.make_async_copy(dst.at[pl.ds(0, CHUNK)], di_v.at[s4],
                              sem_di.at[s4]).wait()
        pltpu.async_copy(ones_v, acc_sh.at[di_v.at[s4]], sem_s.at[p], add=True)
        return carry
    lax.fori_loop(0, n_my, chunk_body, 0, unroll=False)

    @pl.when(n_my >= 1)
    def _():
        pf = lax.rem(n_my - 1, 2)
        pltpu.make_async_copy(ones_v, acc_sh.at[di_v.at[lax.rem(n_my - 1, 4)]],
                              sem_s.at[pf]).wait()

    @pl.when(n_my >= 2)
    def _():
        pf = lax.rem(n_my - 2, 2)
        pltpu.make_async_copy(ones_v, acc_sh.at[di_v.at[lax.rem(n_my - 2, 4)]],
                              sem_s.at[pf]).wait()

    plsc.subcore_barrier()
    pltpu.sync_copy(acc_sh.at[pl.ds(r0, rpt)],
                    deg_o.at[pl.ds(cid * n_pad + r0, rpt)])


def _make_deg_kernel(n_pad, n_edges):
    mesh = plsc.VectorSubcoreMesh(core_axis_name="c", subcore_axis_name="s")
    return pl.kernel(
        functools.partial(_deg_sc_body, n_pad, n_edges),
        out_type=jax.ShapeDtypeStruct((NC * n_pad, 128), jnp.float32),
        mesh=mesh,
        scratch_types=[
            pltpu.VMEM((4, CHUNK), jnp.int32),
            pltpu.VMEM((CHUNK, 128), jnp.float32),
            pltpu.VMEM_SHARED((n_pad, 128), jnp.float32),
            pltpu.SemaphoreType.DMA((2,)),
            pltpu.SemaphoreType.DMA((4,)),
        ],
        name="degree_sc",
    )


REPEATS = 4


def kernel(x, x_mask, edge_attr, pos, sampling_points,
           W_enc, b_enc, W_pos, b_pos, W_msg, b_msg, W_upd, b_upd,
           W_dec, b_dec, edge_index, batch):
    f32 = jnp.float32
    n = x.shape[0]
    e = edge_index.shape[1]
    H = W_enc.shape[1]

    h0 = jnp.concatenate([x, x_mask], axis=1)
    batch2 = batch[:, None]
    batch_r = batch[None, :]
    bc2 = x_mask[:, 1:2]
    src = edge_index[0]
    dst = edge_index[1]
    Wm_s = W_msg[:H]
    Wm_d = W_msg[H:2 * H]
    Wm_e = W_msg[2 * H:2 * H + 4]
    Wm_p = W_msg[2 * H + 4:]
    W1 = W_upd[:H]
    W2 = W_upd[H:2 * H]
    W3 = W_upd[2 * H:3 * H]
    W4 = W_upd[3 * H:]
    n_pad = -(-n // (8 * NS)) * (8 * NS)
    z128 = jnp.zeros((n_pad, 128), f32)

    nf = jax.ShapeDtypeStruct((n, H), f32)
    gf = jax.ShapeDtypeStruct((G, H), f32)
    h, P, oh, ohnt, crow, xbc4b, A2, B2 = pl.pallas_call(
        _pre_body,
        out_shape=(nf, nf, jax.ShapeDtypeStruct((n, G), f32),
                   jax.ShapeDtypeStruct((G, n), f32), gf, gf, nf, nf),
        name="pre_tc",
    )(h0, batch2, batch_r, bc2, pos, W_enc, b_enc[None, :], Wm_s, Wm_d, Wm_p,
      W3, W4, b_upd[None, :])

    eb = 3200
    C = pl.pallas_call(
        _c_body,
        grid=(e // eb,),
        in_specs=[pl.BlockSpec((eb, 4), lambda i: (i, 0)),
                  pl.BlockSpec((4, H), lambda i: (0, 0)),
                  pl.BlockSpec((1, H), lambda i: (0, 0))],
        out_specs=pl.BlockSpec((eb, H), lambda i: (i, 0)),
        out_shape=jax.ShapeDtypeStruct((e, H), f32),
        name="edge_const_tc",
    )(edge_attr, Wm_e, b_msg[None, :])

    edge_k = _make_edge_kernel(n_pad, e)
    degp = _make_deg_kernel(n_pad, e)(dst, z128)
    d0 = degp[:n, :1]
    d1 = degp[n_pad:n_pad + n, :1]
    upd = pl.pallas_call(
        _upd_body,
        out_shape=(nf, nf, nf, gf),
        name="update_tc",
    )

    for _ in range(REPEATS):
        aggp = edge_k(A2, B2, C, src, dst, z128)
        h, A2, B2, crow = upd(
            h, aggp[:n], aggp[n_pad:n_pad + n], d0, d1,
            oh, ohnt, crow, P, xbc4b, W1, W2, Wm_s, Wm_d, W3)

    U, nodes = pl.pallas_call(
        _epi_body,
        out_shape=(jax.ShapeDtypeStruct((sampling_points.shape[0], 4), f32),
                   jax.ShapeDtypeStruct((n, 4), f32)),
        name="epi_tc",
    )(h, sampling_points, W_dec, b_dec[None, :], W_pos, b_pos[None, :])
    return (U, nodes)


# speedup vs baseline: 7.0704x; 1.0010x over previous
import functools

import jax
import jax.numpy as jnp
from jax import lax
from jax.experimental import pallas as pl
from jax.experimental.pallas import tpu as pltpu
from jax.experimental.pallas import tpu_sc as plsc

G = 16
NC = 2
NS = 16
CHUNK = 64


def _pre_body(h0, batch2, batch_r, bc2, pos, W_enc, b_enc, Wm_s, Wm_d, Wm_p,
              W3, W4, b_upd,
              h_o, P_o, oh_o, ohnt_o, crow_o, xbc4b_o, A_o, B_o):
    f32 = jnp.float32
    h = jnp.maximum(jnp.dot(h0[...], W_enc[...],
                            preferred_element_type=f32) + b_enc[...], 0.0)
    n = h.shape[0]
    iota_cols = lax.broadcasted_iota(jnp.int32, (n, G), 1)
    oh = (batch2[...] == iota_cols).astype(f32)
    iota_rows = lax.broadcasted_iota(jnp.int32, (G, n), 0)
    oht = (batch_r[...] == iota_rows).astype(f32)
    cnt = jnp.sum(oht, axis=1, keepdims=True)
    ohnt = oht / jnp.maximum(cnt, 1.0)
    bc = (bc2[...] > 0.5).astype(f32)
    ohbct = oht * jnp.reshape(bc, (1, n))
    cnt_bc = jnp.maximum(jnp.sum(ohbct, axis=1, keepdims=True), 1.0)
    x_bc = jnp.dot(ohbct, h, preferred_element_type=f32) / cnt_bc
    xg = jnp.dot(ohnt, h, preferred_element_type=f32)
    xbc4b = jnp.dot(x_bc, W4[...], preferred_element_type=f32) + b_upd[...]
    crow = jnp.dot(xg, W3[...], preferred_element_type=f32) + xbc4b
    P = jnp.dot(pos[...], Wm_p[...], preferred_element_type=f32)
    h_o[...] = h
    P_o[...] = P
    oh_o[...] = oh
    ohnt_o[...] = ohnt
    crow_o[...] = crow
    xbc4b_o[...] = xbc4b
    A_o[...] = jnp.dot(h, Wm_s[...], preferred_element_type=f32) - P
    B_o[...] = jnp.dot(h, Wm_d[...], preferred_element_type=f32) + P


def _c_body(ea, Wm_e, b_msg, c_o):
    c_o[...] = (jnp.dot(ea[...], Wm_e[...], preferred_element_type=jnp.float32)
                + b_msg[...])


def _upd_body(h, a0, a1, d0, d1, oh, ohnt, crow, P, xbc4b,
              W1, W2, Wm_s, Wm_d, W3,
              h_o, A_o, B_o, crow_o):
    f32 = jnp.float32
    deg = jnp.maximum(d0[...] + d1[...], 1.0)
    agg = (a0[...] + a1[...]) / deg
    u = jnp.maximum(
        jnp.dot(h[...], W1[...], preferred_element_type=f32)
        + jnp.dot(agg, W2[...], preferred_element_type=f32)
        + jnp.dot(oh[...], crow[...], preferred_element_type=f32), 0.0)
    h2 = h[...] + u
    xg = jnp.dot(ohnt[...], h2, preferred_element_type=f32)
    h_o[...] = h2
    A_o[...] = jnp.dot(h2, Wm_s[...], preferred_element_type=f32) - P[...]
    B_o[...] = jnp.dot(h2, Wm_d[...], preferred_element_type=f32) + P[...]
    crow_o[...] = jnp.dot(xg, W3[...], preferred_element_type=f32) + xbc4b[...]


def _epi_body(h, sp, W_dec, b_dec, W_pos, b_pos, u_o, nodes_o):
    f32 = jnp.float32
    nodes_o[...] = (jnp.dot(h[...], W_dec[...], preferred_element_type=f32)
                    + b_dec[...])
    es = jnp.maximum(jnp.dot(sp[...], W_pos[...],
                             preferred_element_type=f32) + b_pos[...], 0.0)
    u_o[...] = jnp.dot(es, W_dec[...], preferred_element_type=f32) + b_dec[...]


def _chunk_ranges(n_edges):
    total_chunks = n_edges // CHUNK
    cpw = -(-total_chunks // (NC * NS))
    return total_chunks, cpw


def _edge_sc_body(n_pad, n_edges,
                  A2, B2, C, src, dst, z128,
                  agg_o,
                  si_v, di_v, a_v, b_v, c_v,
                  acc_sh, sem_a, sem_b, sem_c, sem_s, sem_si, sem_di):
    cid = lax.axis_index("c")
    sid = lax.axis_index("s")
    wid = sid * NC + cid
    rpt = n_pad // NS
    r0 = sid * rpt

    total_chunks, cpw = _chunk_ranges(n_edges)
    start = wid * cpw
    n_my = jnp.maximum(jnp.minimum(cpw, total_chunks - start), 0)

    pltpu.sync_copy(z128.at[pl.ds(r0, rpt)], acc_sh.at[pl.ds(r0, rpt)])
    plsc.subcore_barrier()

    def idx_fetch(ci):
        s4 = lax.rem(ci, 4)
        ebase = (start + ci) * CHUNK
        pltpu.async_copy(src.at[pl.ds(ebase, CHUNK)], si_v.at[s4],
                         sem_si.at[s4])
        pltpu.async_copy(dst.at[pl.ds(ebase, CHUNK)], di_v.at[s4],
                         sem_di.at[s4])

    def idx_wait(ci):
        s4 = lax.rem(ci, 4)
        pltpu.make_async_copy(src.at[pl.ds(0, CHUNK)], si_v.at[s4],
                              sem_si.at[s4]).wait()
        pltpu.make_async_copy(dst.at[pl.ds(0, CHUNK)], di_v.at[s4],
                              sem_di.at[s4]).wait()

    def stage(slot, ci):
        ebase = (start + ci) * CHUNK
        s4 = lax.rem(ci, 4)
        pltpu.async_copy(A2.at[si_v.at[s4]], a_v.at[slot], sem_a.at[slot])
        pltpu.async_copy(B2.at[di_v.at[s4]], b_v.at[slot], sem_b.at[slot])
        pltpu.async_copy(C.at[pl.ds(ebase, CHUNK)], c_v.at[slot],
                         sem_c.at[slot])

    @pl.when(n_my > 0)
    def _():
        idx_fetch(0)
        idx_wait(0)
        stage(0, 0)

    @pl.when(n_my > 1)
    def _():
        idx_fetch(1)

    def chunk_body(i, carry):
        p = lax.rem(i, 2)
        q = lax.rem(i + 1, 2)
        ebase = (start + i) * CHUNK

        @pl.when(i >= 1)
        def _():
            pltpu.make_async_copy(c_v.at[q], acc_sh.at[di_v.at[lax.rem(i - 1, 4)]],
                                  sem_s.at[q]).wait()

        @pl.when(i + 2 < n_my)
        def _():
            idx_fetch(i + 2)

        @pl.when(i + 1 < n_my)
        def _():
            idx_wait(i + 1)
            stage(q, i + 1)

        pltpu.make_async_copy(A2.at[si_v.at[lax.rem(i, 4)]], a_v.at[p],
                              sem_a.at[p]).wait()
        pltpu.make_async_copy(B2.at[di_v.at[lax.rem(i, 4)]], b_v.at[p],
                              sem_b.at[p]).wait()
        pltpu.make_async_copy(C.at[pl.ds(ebase, CHUNK)], c_v.at[p],
                              sem_c.at[p]).wait()

        @plsc.parallel_loop(0, CHUNK, unroll=4)
        def row_body(r):
            for j in range(8):
                s = pl.ds(j * 16, 16)
                c_v[p, r, s] = jnp.maximum(
                    a_v[p, r, s] + b_v[p, r, s] + c_v[p, r, s], 0.0)

        pltpu.async_copy(c_v.at[p], acc_sh.at[di_v.at[lax.rem(i, 4)]],
                         sem_s.at[p], add=True)
        return carry
    lax.fori_loop(0, n_my, chunk_body, 0, unroll=False)

    @pl.when(n_my >= 1)
    def _():
        pf = lax.rem(n_my - 1, 2)
        pltpu.make_async_copy(c_v.at[pf],
                              acc_sh.at[di_v.at[lax.rem(n_my - 1, 4)]],
                              sem_s.at[pf]).wait()

    plsc.subcore_barrier()
    pltpu.sync_copy(acc_sh.at[pl.ds(r0, rpt)],
                    agg_o.at[pl.ds(cid * n_pad + r0, rpt)])


def _make_edge_kernel(n_pad, n_edges):
    mesh = plsc.VectorSubcoreMesh(core_axis_name="c", subcore_axis_name="s")
    return pl.kernel(
        functools.partial(_edge_sc_body, n_pad, n_edges),
        out_type=jax.ShapeDtypeStruct((NC * n_pad, 128), jnp.float32),
        mesh=mesh,
        scratch_types=[
            pltpu.VMEM((4, CHUNK), jnp.int32),
            pltpu.VMEM((4, CHUNK), jnp.int32),
            pltpu.VMEM((2, CHUNK, 128), jnp.float32),
            pltpu.VMEM((2, CHUNK, 128), jnp.float32),
            pltpu.VMEM((2, CHUNK, 128), jnp.float32),
            pltpu.VMEM_SHARED((n_pad, 128), jnp.float32),
            pltpu.SemaphoreType.DMA((2,)),
            pltpu.SemaphoreType.DMA((2,)),
            pltpu.SemaphoreType.DMA((2,)),
            pltpu.SemaphoreType.DMA((2,)),
            pltpu.SemaphoreType.DMA((4,)),
            pltpu.SemaphoreType.DMA((4,)),
        ],
        name="edge_messages_sc",
    )


def _deg_sc_body(n_pad, n_edges,
                 dst, z128,
                 deg_o,
                 di_v, ones_v, acc_sh, sem_s, sem_di):
    cid = lax.axis_index("c")
    sid = lax.axis_index("s")
    wid = sid * NC + cid
    rpt = n_pad // NS
    r0 = sid * rpt

    total_chunks, cpw = _chunk_ranges(n_edges)
    start = wid * cpw
    n_my = jnp.maximum(jnp.minimum(cpw, total_chunks - start), 0)

    pltpu.sync_copy(z128.at[pl.ds(r0, rpt)], acc_sh.at[pl.ds(r0, rpt)])
    one = jnp.ones((16,), jnp.float32)

    def fill_body(r, carry):
        for j in range(8):
            ones_v[r, pl.ds(j * 16, 16)] = one
        return carry
    lax.fori_loop(0, CHUNK, fill_body, 0, unroll=False)

    def idx_fetch(ci):
        s4 = lax.rem(ci, 4)
        pltpu.async_copy(dst.at[pl.ds((start + ci) * CHUNK, CHUNK)],
                         di_v.at[s4], sem_di.at[s4])

    @pl.when(n_my > 0)
    def _():
        idx_fetch(0)

    @pl.when(n_my > 1)
    def _():
        idx_fetch(1)
    plsc.subcore_barrier()

    def chunk_body(i, carry):
        p = lax.rem(i, 2)
        s4 = lax.rem(i, 4)

        @pl.when(i >= 2)
        def _():
            pltpu.make_async_copy(ones_v, acc_sh.at[di_v.at[lax.rem(i - 2, 4)]],
                                  sem_s.at[p]).wait()

        @pl.when(i + 2 < n_my)
        def _():
            idx_fetch(i + 2)

        pltpu.make_async_copy(dst.at[pl.ds(0, CHUNK)], di_v.at[s4],
                              sem_di.at[s4]).wait()
        pltpu.async_copy(ones_v, acc_sh.at[di_v.at[s4]], sem_s.at[p], add=True)
        return carry
    lax.fori_loop(0, n_my, chunk_body, 0, unroll=False)

    @pl.when(n_my >= 1)
    def _():
        pf = lax.rem(n_my - 1, 2)
        pltpu.make_async_copy(ones_v, acc_sh.at[di_v.at[lax.rem(n_my - 1, 4)]],
                              sem_s.at[pf]).wait()

    @pl.when(n_my >= 2)
    def _():
        pf = lax.rem(n_my - 2, 2)
        pltpu.make_async_copy(ones_v, acc_sh.at[di_v.at[lax.rem(n_my - 2, 4)]],
                              sem_s.at[pf]).wait()

    plsc.subcore_barrier()
    pltpu.sync_copy(acc_sh.at[pl.ds(r0, rpt)],
                    deg_o.at[pl.ds(cid * n_pad + r0, rpt)])


def _make_deg_kernel(n_pad, n_edges):
    mesh = plsc.VectorSubcoreMesh(core_axis_name="c", subcore_axis_name="s")
    return pl.kernel(
        functools.partial(_deg_sc_body, n_pad, n_edges),
        out_type=jax.ShapeDtypeStruct((NC * n_pad, 128), jnp.float32),
        mesh=mesh,
        scratch_types=[
            pltpu.VMEM((4, CHUNK), jnp.int32),
            pltpu.VMEM((CHUNK, 128), jnp.float32),
            pltpu.VMEM_SHARED((n_pad, 128), jnp.float32),
            pltpu.SemaphoreType.DMA((2,)),
            pltpu.SemaphoreType.DMA((4,)),
        ],
        name="degree_sc",
    )


REPEATS = 4


def kernel(x, x_mask, edge_attr, pos, sampling_points,
           W_enc, b_enc, W_pos, b_pos, W_msg, b_msg, W_upd, b_upd,
           W_dec, b_dec, edge_index, batch):
    f32 = jnp.float32
    n = x.shape[0]
    e = edge_index.shape[1]
    H = W_enc.shape[1]

    h0 = jnp.concatenate([x, x_mask], axis=1)
    batch2 = batch[:, None]
    batch_r = batch[None, :]
    bc2 = x_mask[:, 1:2]
    src = edge_index[0]
    dst = edge_index[1]
    Wm_s = W_msg[:H]
    Wm_d = W_msg[H:2 * H]
    Wm_e = W_msg[2 * H:2 * H + 4]
    Wm_p = W_msg[2 * H + 4:]
    W1 = W_upd[:H]
    W2 = W_upd[H:2 * H]
    W3 = W_upd[2 * H:3 * H]
    W4 = W_upd[3 * H:]
    n_pad = -(-n // (8 * NS)) * (8 * NS)
    z128 = jnp.zeros((n_pad, 128), f32)

    nf = jax.ShapeDtypeStruct((n, H), f32)
    gf = jax.ShapeDtypeStruct((G, H), f32)
    h, P, oh, ohnt, crow, xbc4b, A2, B2 = pl.pallas_call(
        _pre_body,
        out_shape=(nf, nf, jax.ShapeDtypeStruct((n, G), f32),
                   jax.ShapeDtypeStruct((G, n), f32), gf, gf, nf, nf),
        name="pre_tc",
    )(h0, batch2, batch_r, bc2, pos, W_enc, b_enc[None, :], Wm_s, Wm_d, Wm_p,
      W3, W4, b_upd[None, :])

    eb = 3200
    C = pl.pallas_call(
        _c_body,
        grid=(e // eb,),
        in_specs=[pl.BlockSpec((eb, 4), lambda i: (i, 0)),
                  pl.BlockSpec((4, H), lambda i: (0, 0)),
                  pl.BlockSpec((1, H), lambda i: (0, 0))],
        out_specs=pl.BlockSpec((eb, H), lambda i: (i, 0)),
        out_shape=jax.ShapeDtypeStruct((e, H), f32),
        name="edge_const_tc",
    )(edge_attr, Wm_e, b_msg[None, :])

    edge_k = _make_edge_kernel(n_pad, e)
    degp = _make_deg_kernel(n_pad, e)(dst, z128)
    d0 = degp[:n, :1]
    d1 = degp[n_pad:n_pad + n, :1]
    upd = pl.pallas_call(
        _upd_body,
        out_shape=(nf, nf, nf, gf),
        name="update_tc",
    )

    for _ in range(REPEATS):
        aggp = edge_k(A2, B2, C, src, dst, z128)
        h, A2, B2, crow = upd(
            h, aggp[:n], aggp[n_pad:n_pad + n], d0, d1,
            oh, ohnt, crow, P, xbc4b, W1, W2, Wm_s, Wm_d, W3)

    U, nodes = pl.pallas_call(
        _epi_body,
        out_shape=(jax.ShapeDtypeStruct((sampling_points.shape[0], 4), f32),
                   jax.ShapeDtypeStruct((n, 4), f32)),
        name="epi_tc",
    )(h, sampling_points, W_dec, b_dec[None, :], W_pos, b_pos[None, :])
    return (U, nodes)
